# baseline (speedup 1.0000x reference)
import functools

import jax
import jax.numpy as jnp
from jax import lax
from jax.experimental import pallas as pl
from jax.experimental.pallas import tpu as pltpu
from jax.experimental.pallas import tpu_sc as plsc

NUM_TOKENS = 16384
D_MODEL = 2048
NUM_EXPERTS = 16
TOP_K = 2

BT = 2048

NC = 2
NS = 16
LANES = 16
NW = NC * NS
TPW = NUM_TOKENS // NW
GROUPS = TPW // LANES


def _logits_block(x_ref, w_ref, out_ref):
    logits = jnp.dot(
        x_ref[...], w_ref[...].T, preferred_element_type=jnp.float32
    )
    out_ref[...] = logits.T


def _tc_logits_t(x, W):
    grid = (NUM_TOKENS // BT,)
    return pl.pallas_call(
        _logits_block,
        grid=grid,
        in_specs=[
            pl.BlockSpec((BT, D_MODEL), lambda i: (i, 0)),
            pl.BlockSpec((NUM_EXPERTS, D_MODEL), lambda i: (0, 0)),
        ],
        out_specs=pl.BlockSpec((NUM_EXPERTS, BT), lambda i: (0, i)),
        out_shape=jax.ShapeDtypeStruct((NUM_EXPERTS, NUM_TOKENS), jnp.float32),
        compiler_params=pltpu.CompilerParams(
            dimension_semantics=("parallel",),
        ),
    )(x, W)


def _sc_route(logits_t):
    mesh = plsc.VectorSubcoreMesh(core_axis_name="c", subcore_axis_name="s")

    @functools.partial(
        pl.kernel,
        mesh=mesh,
        out_type=[
            jax.ShapeDtypeStruct((TOP_K, NUM_TOKENS), jnp.float32),
            jax.ShapeDtypeStruct((TOP_K, NUM_TOKENS), jnp.int32),
        ],
        scratch_types=[
            pltpu.VMEM((NUM_EXPERTS, TPW), jnp.float32),
            pltpu.VMEM((TPW,), jnp.float32),
            pltpu.VMEM((TPW,), jnp.float32),
            pltpu.VMEM((TPW,), jnp.int32),
            pltpu.VMEM((TPW,), jnp.int32),
        ],
    )
    def route(lt_hbm, w_hbm, idx_hbm, lbuf, w1buf, w2buf, i1buf, i2buf):
        wid = lax.axis_index("s") * NC + lax.axis_index("c")
        base = wid * TPW
        pltpu.sync_copy(lt_hbm.at[:, pl.ds(base, TPW)], lbuf)

        zeros = jnp.zeros((LANES,), jnp.int32)

        def body(g, carry):
            off = g * LANES
            max1 = lbuf[0, pl.ds(off, LANES)]
            idx1 = zeros
            max2 = jnp.full((LANES,), -jnp.inf, jnp.float32)
            idx2 = zeros
            for e in range(1, NUM_EXPERTS):
                v = lbuf[e, pl.ds(off, LANES)]
                gt1 = v > max1
                gt2 = v > max2
                idx2 = jnp.where(gt1, idx1, jnp.where(gt2, e, idx2))
                max2 = jnp.where(gt1, max1, jnp.where(gt2, v, max2))
                idx1 = jnp.where(gt1, e, idx1)
                max1 = jnp.where(gt1, v, max1)
            e2 = jnp.exp(max2 - max1)
            s = 1.0 + e2
            w1buf[pl.ds(off, LANES)] = 1.0 / s
            w2buf[pl.ds(off, LANES)] = e2 / s
            i1buf[pl.ds(off, LANES)] = idx1
            i2buf[pl.ds(off, LANES)] = idx2
            return carry

        lax.fori_loop(0, GROUPS, body, 0)
        pltpu.sync_copy(w1buf, w_hbm.at[0, pl.ds(base, TPW)])
        pltpu.sync_copy(w2buf, w_hbm.at[1, pl.ds(base, TPW)])
        pltpu.sync_copy(i1buf, idx_hbm.at[0, pl.ds(base, TPW)])
        pltpu.sync_copy(i2buf, idx_hbm.at[1, pl.ds(base, TPW)])

    return route(logits_t)


def kernel(x, W):
    logits_t = _tc_logits_t(x, W)
    w_pl, idx_pl = _sc_route(logits_t)
    return (w_pl.T, idx_pl.T)


# speedup vs baseline: 1.4747x; 1.4747x over previous
import jax
import jax.numpy as jnp
from jax.experimental import pallas as pl
from jax.experimental.pallas import tpu as pltpu

NUM_TOKENS = 16384
D_MODEL = 2048
NUM_EXPERTS = 16
TOP_K = 2

BT = 1024


def _gate_block(x_ref, w_ref, w_out_ref, idx_out_ref):
    logits = jnp.dot(
        x_ref[...], w_ref[...].T, preferred_element_type=jnp.float32
    )
    lt = logits.T
    iota = jax.lax.broadcasted_iota(jnp.int32, lt.shape, 0)
    l1 = jnp.max(lt, axis=0, keepdims=True)
    i1 = jnp.min(
        jnp.where(lt == l1, iota, NUM_EXPERTS), axis=0, keepdims=True
    )
    masked = jnp.where(iota == i1, -jnp.inf, lt)
    l2 = jnp.max(masked, axis=0, keepdims=True)
    i2 = jnp.min(
        jnp.where(masked == l2, iota, NUM_EXPERTS), axis=0, keepdims=True
    )
    e2 = jnp.exp(l2 - l1)
    s = 1.0 + e2
    w_out_ref[0:1, :] = 1.0 / s
    w_out_ref[1:2, :] = e2 / s
    idx_out_ref[0:1, :] = i1
    idx_out_ref[1:2, :] = i2


def kernel(x, W):
    grid = (NUM_TOKENS // BT,)
    w_pl, idx_pl = pl.pallas_call(
        _gate_block,
        grid=grid,
        in_specs=[
            pl.BlockSpec((BT, D_MODEL), lambda i: (i, 0)),
            pl.BlockSpec((NUM_EXPERTS, D_MODEL), lambda i: (0, 0)),
        ],
        out_specs=[
            pl.BlockSpec((TOP_K, BT), lambda i: (0, i)),
            pl.BlockSpec((TOP_K, BT), lambda i: (0, i)),
        ],
        out_shape=[
            jax.ShapeDtypeStruct((TOP_K, NUM_TOKENS), jnp.float32),
            jax.ShapeDtypeStruct((TOP_K, NUM_TOKENS), jnp.int32),
        ],
        compiler_params=pltpu.CompilerParams(
            dimension_semantics=("parallel",),
        ),
    )(x, W)
    return (w_pl.T, idx_pl.T)
